# Initial kernel scaffold; baseline (speedup 1.0000x reference)
import jax
import jax.numpy as jnp
from jax.experimental import pallas as pl


def kernel(pos_undirected, pos_directed, params, nfreq, seed, efreq, edge_index):
    raise NotImplementedError("write your pallas kernel here")


# speedup vs baseline: 9.0266x; 9.0266x over previous
import functools

import jax
import jax.numpy as jnp
from jax import lax
from jax.experimental import pallas as pl
from jax.experimental.pallas import tpu as pltpu
from jax.experimental.pallas import tpu_sc as plsc

N = 10000
E = 160000
P = 16
D = 32
MAX_NF = 8
MAX_EF = 8
NF = MAX_EF + 1
T_MP = 3
T_S2S = 6

NC, NS = 2, 16
CHUNK = 128
EPAD = 163840
EROWS = EPAD // CHUNK
RPW = EROWS // (NC * NS)
NROWS = 10112
RPT = NROWS // NS
NBLK = 2000
ER2 = E // CHUNK

_f32 = jnp.float32


def _const_body(ef, w1, b1, w2, b2, src, efq, ewtab, eidx):
    v = jax.nn.relu(jnp.dot(ef[...], w1[...], preferred_element_type=_f32) + b1[...])
    ewtab[...] = jnp.dot(v, w2[...], preferred_element_type=_f32) + b2[...]
    eidx[...] = src[...] * NF + jnp.clip(efq[...], 0, MAX_EF)


_const_call = pl.pallas_call(
    _const_body,
    out_shape=[jax.ShapeDtypeStruct((NF, D * D), _f32),
               jax.ShapeDtypeStruct((ER2, CHUNK), jnp.int32)],
)


def _init_body(pu, pd, nfq, sd, emb, w0a, w0b, w0c, w0d, w0e, b0, wbig,
               out0, outp):
    nfi = nfq[...]
    oh = (lax.broadcasted_iota(jnp.int32, (NBLK, NF), 1)
          == jnp.clip(nfi, 0, MAX_NF)).astype(_f32)
    nemb = jnp.dot(oh, emb[...], preferred_element_type=_f32)
    x = (jnp.dot(pu[...], w0a[...], preferred_element_type=_f32)
         + jnp.dot(pd[...], w0b[...], preferred_element_type=_f32)
         + jnp.dot(nemb, w0c[...], preferred_element_type=_f32)
         + sd[...] * w0d[...]
         + (nfi.astype(_f32) * (1.0 / MAX_NF)) * w0e[...] + b0[...])
    o = jax.nn.relu(x)
    out0[...] = o
    outp[...] = jnp.dot(o, wbig[...], preferred_element_type=_f32)


def _whole(shape):
    return pl.BlockSpec(shape, lambda i: (0,) * len(shape))


_init_call = pl.pallas_call(
    _init_body,
    grid=(N // NBLK,),
    in_specs=[pl.BlockSpec((NBLK, P), lambda i: (i, 0)),
              pl.BlockSpec((NBLK, P), lambda i: (i, 0)),
              pl.BlockSpec((NBLK, 1), lambda i: (i, 0)),
              pl.BlockSpec((NBLK, 1), lambda i: (i, 0)),
              _whole((NF, D)),
              _whole((P, D)), _whole((P, D)), _whole((D, D)),
              _whole((1, D)), _whole((1, D)), _whole((1, D)),
              _whole((D, NF * D))],
    out_specs=[pl.BlockSpec((NBLK, D), lambda i: (i, 0)),
               pl.BlockSpec((NBLK, NF * D), lambda i: (i, 0))],
    out_shape=[jax.ShapeDtypeStruct((N, D), _f32),
               jax.ShapeDtypeStruct((N, NF * D), _f32)],
)


def _mp_body(outp_hbm, eidx_hbm, dst_hbm, aggs_hbm,
             zbuf, eidx_v, dst_v, rows_v, agg_sh, sem):
    cid = lax.axis_index("c")
    sid = lax.axis_index("s")

    def zr(i, c):
        zbuf[i, pl.ds(0, 16)] = jnp.zeros((16,), _f32)
        zbuf[i, pl.ds(16, 16)] = jnp.zeros((16,), _f32)
        return c

    lax.fori_loop(0, RPT, zr, 0)
    pltpu.sync_copy(zbuf, agg_sh.at[pl.ds(sid * RPT, RPT)])
    plsc.subcore_barrier()

    base = cid * (EROWS // NC) + sid * RPW
    pltpu.sync_copy(eidx_hbm.at[pl.ds(base, RPW)], eidx_v)
    pltpu.sync_copy(dst_hbm.at[pl.ds(base, RPW)], dst_v)

    def step(j, c):
        pltpu.async_copy(outp_hbm.at[eidx_v.at[j]], rows_v, sem).wait()
        pltpu.sync_copy(rows_v, agg_sh.at[dst_v.at[j]], add=True)
        return c

    lax.fori_loop(0, RPW, step, 0)
    plsc.subcore_barrier()
    pltpu.sync_copy(agg_sh.at[pl.ds(sid * RPT, RPT)], zbuf)
    pltpu.sync_copy(zbuf, aggs_hbm.at[pl.ds(cid * NROWS + sid * RPT, RPT)])


_mp_call = pl.kernel(
    _mp_body,
    out_type=jax.ShapeDtypeStruct((NC * NROWS, D), _f32),
    mesh=plsc.VectorSubcoreMesh(core_axis_name="c", subcore_axis_name="s",
                                num_cores=NC, num_subcores=NS),
    scratch_types=[pltpu.VMEM((RPT, D), _f32),
                   pltpu.VMEM((RPW, CHUNK), jnp.int32),
                   pltpu.VMEM((RPW, CHUNK), jnp.int32),
                   pltpu.VMEM((CHUNK, D), _f32),
                   pltpu.VMEM_SHARED((NROWS, D), _f32),
                   pltpu.SemaphoreType.DMA],
    compiler_params=pltpu.CompilerParams(use_tc_tiling_on_sc=False),
)


def _gru_body(a0, a1, h, cb, wir, wiz, win, bir, biz, bin_, whr, whz, whn,
              bhr, bhz, bhn, wbig, hout, outp):
    hv = h[...]
    m = jax.nn.relu(a0[...] + a1[...] + cb[...])
    r = jax.nn.sigmoid(jnp.dot(m, wir[...], preferred_element_type=_f32) + bir[...]
                       + jnp.dot(hv, whr[...], preferred_element_type=_f32) + bhr[...])
    z = jax.nn.sigmoid(jnp.dot(m, wiz[...], preferred_element_type=_f32) + biz[...]
                       + jnp.dot(hv, whz[...], preferred_element_type=_f32) + bhz[...])
    hn = jnp.dot(hv, whn[...], preferred_element_type=_f32) + bhn[...]
    n_ = jnp.tanh(jnp.dot(m, win[...], preferred_element_type=_f32) + bin_[...] + r * hn)
    hnew = (1.0 - z) * n_ + z * hv
    hout[...] = hnew
    outp[...] = jnp.dot(hnew, wbig[...], preferred_element_type=_f32)


_gru_call = pl.pallas_call(
    _gru_body,
    grid=(N // NBLK,),
    in_specs=[pl.BlockSpec((NBLK, D), lambda i: (i, 0)),
              pl.BlockSpec((NBLK, D), lambda i: (i, 0)),
              pl.BlockSpec((NBLK, D), lambda i: (i, 0)),
              _whole((1, D)),
              _whole((D, D)), _whole((D, D)), _whole((D, D)),
              _whole((1, D)), _whole((1, D)), _whole((1, D)),
              _whole((D, D)), _whole((D, D)), _whole((D, D)),
              _whole((1, D)), _whole((1, D)), _whole((1, D)),
              _whole((D, NF * D))],
    out_specs=[pl.BlockSpec((NBLK, D), lambda i: (i, 0)),
               pl.BlockSpec((NBLK, NF * D), lambda i: (i, 0))],
    out_shape=[jax.ShapeDtypeStruct((N, D), _f32),
               jax.ShapeDtypeStruct((N, NF * D), _f32)],
)


def _s2s_body(h_ref, w0q, w0r, u0, b0, w1g, u1, b1, w2g, u2, b2,
              l1q, l1r, l1b, l2w, l2b, out_ref):
    x = h_ref[...]
    w0qv, w0rv, u0v, b0v = w0q[...], w0r[...], u0[...], b0[...]
    w1v, u1v, b1v = w1g[...], u1[...], b1[...]
    w2v, u2v, b2v = w2g[...], u2[...], b2[...]
    zero = jnp.zeros((1, D), _f32)

    def gates(xs, ws, u, b, hp):
        out = []
        for g in range(4):
            acc = b[g] + jnp.dot(hp, u[g], preferred_element_type=_f32)
            for xx, w in zip(xs, ws):
                acc = acc + jnp.dot(xx, w[g], preferred_element_type=_f32)
            out.append(acc)
        return out

    def it(t, carry):
        q, r, h0, c0, h1, c1, h2, c2 = carry
        gi, gf, gg, go = gates([q, r], [w0qv, w0rv], u0v, b0v, h0)
        c0 = jax.nn.sigmoid(gf) * c0 + jax.nn.sigmoid(gi) * jnp.tanh(gg)
        h0 = jax.nn.sigmoid(go) * jnp.tanh(c0)
        gi, gf, gg, go = gates([h0], [w1v], u1v, b1v, h1)
        c1 = jax.nn.sigmoid(gf) * c1 + jax.nn.sigmoid(gi) * jnp.tanh(gg)
        h1 = jax.nn.sigmoid(go) * jnp.tanh(c1)
        gi, gf, gg, go = gates([h1], [w2v], u2v, b2v, h2)
        c2 = jax.nn.sigmoid(gf) * c2 + jax.nn.sigmoid(gi) * jnp.tanh(gg)
        h2 = jax.nn.sigmoid(go) * jnp.tanh(c2)
        q = h2
        e = jnp.sum(x * q, axis=1, keepdims=True)
        a = jnp.exp(e - jnp.max(e))
        r = jnp.sum(a * x, axis=0, keepdims=True) / jnp.sum(a)
        return (q, r, h0, c0, h1, c1, h2, c2)

    q, r = lax.fori_loop(0, T_S2S, it, (zero,) * 8)[:2]
    y = jax.nn.relu(jnp.dot(q, l1q[...], preferred_element_type=_f32)
                    + jnp.dot(r, l1r[...], preferred_element_type=_f32) + l1b[...])
    y = jnp.dot(y, l2w[...], preferred_element_type=_f32) + l2b[...]
    out_ref[...] = jnp.broadcast_to(y, (8, D))


_s2s_call = pl.pallas_call(
    _s2s_body,
    out_shape=jax.ShapeDtypeStruct((8, D), _f32),
)


def kernel(pos_undirected, pos_directed, params, nfreq, seed, efreq, edge_index):
    p = params
    src = edge_index[0].astype(jnp.int32)
    dst = edge_index[1].astype(jnp.int32)
    efq = efreq.astype(jnp.int32)

    ef_feat = jnp.concatenate(
        [p['edge_freq_emb'], (jnp.arange(NF, dtype=_f32) / MAX_EF)[:, None]], axis=1)
    ewtab, eidx2d = _const_call(
        ef_feat, p['edge_W1'], p['edge_b1'][None], p['edge_W2'], p['edge_b2'][None],
        src.reshape(ER2, CHUNK), efq.reshape(ER2, CHUNK))
    wbig = ewtab.reshape(NF, D, D).transpose(1, 0, 2).reshape(D, NF * D)
    eidx = jnp.concatenate(
        [eidx2d.reshape(-1), jnp.zeros((EPAD - E,), jnp.int32)]).reshape(EROWS, CHUNK)
    dstp = jnp.concatenate(
        [dst, jnp.full((EPAD - E,), N, jnp.int32)]).reshape(EROWS, CHUNK)

    w0 = p['lin0_W']
    h, outp = _init_call(
        pos_undirected, pos_directed, nfreq.astype(jnp.int32)[:, None],
        seed.astype(_f32)[:, None], p['node_freq_emb'],
        w0[0:P], w0[P:2 * P], w0[2 * P:2 * P + D],
        w0[2 * P + D:2 * P + D + 1], w0[2 * P + D + 1:2 * P + D + 2],
        p['lin0_b'][None], wbig)

    wih, bih, whh, bhh = p['gru_Wih'], p['gru_bih'], p['gru_Whh'], p['gru_bhh']
    for _ in range(T_MP):
        aggs = _mp_call(outp.reshape(N * NF, D), eidx, dstp)
        h, outp = _gru_call(
            aggs[0:N], aggs[NROWS:NROWS + N], h, p['conv_bias'][None],
            wih[:, 0:D], wih[:, D:2 * D], wih[:, 2 * D:3 * D],
            bih[None, 0:D], bih[None, D:2 * D], bih[None, 2 * D:3 * D],
            whh[:, 0:D], whh[:, D:2 * D], whh[:, 2 * D:3 * D],
            bhh[None, 0:D], bhh[None, D:2 * D], bhh[None, 2 * D:3 * D],
            wbig)

    def stkw(wm, r0, r1):
        return jnp.stack([wm[r0:r1, g * D:(g + 1) * D] for g in range(4)])

    def stkb(l):
        b = p['lstm%d_bih' % l] + p['lstm%d_bhh' % l]
        return jnp.stack([b[None, g * D:(g + 1) * D] for g in range(4)])

    y8 = _s2s_call(
        h,
        stkw(p['lstm0_Wih'], 0, D), stkw(p['lstm0_Wih'], D, 2 * D),
        stkw(p['lstm0_Whh'], 0, D), stkb(0),
        stkw(p['lstm1_Wih'], 0, D), stkw(p['lstm1_Whh'], 0, D), stkb(1),
        stkw(p['lstm2_Wih'], 0, D), stkw(p['lstm2_Whh'], 0, D), stkb(2),
        p['lin1_W'][0:D], p['lin1_W'][D:2 * D], p['lin1_b'][None],
        p['lin2_W'], p['lin2_b'][None])
    return y8[0:1]


# speedup vs baseline: 10.3795x; 1.1499x over previous
import functools

import jax
import jax.numpy as jnp
from jax import lax
from jax.experimental import pallas as pl
from jax.experimental.pallas import tpu as pltpu
from jax.experimental.pallas import tpu_sc as plsc

N = 10000
E = 160000
P = 16
D = 32
MAX_NF = 8
MAX_EF = 8
NF = MAX_EF + 1
T_MP = 3
T_S2S = 6

NC, NS = 2, 16
CHUNK = 128
EPAD = 163840
EROWS = EPAD // CHUNK
RPW = EROWS // (NC * NS)
NROWS = 10112
RPT = NROWS // NS
NBLK = 2000
ER2 = E // CHUNK

_f32 = jnp.float32


def _const_body(ef, w1, b1, w2, b2, src, efq, ewtab, eidx):
    v = jax.nn.relu(jnp.dot(ef[...], w1[...], preferred_element_type=_f32) + b1[...])
    ewtab[...] = jnp.dot(v, w2[...], preferred_element_type=_f32) + b2[...]
    eidx[...] = src[...] * NF + jnp.clip(efq[...], 0, MAX_EF)


_const_call = pl.pallas_call(
    _const_body,
    out_shape=[jax.ShapeDtypeStruct((NF, D * D), _f32),
               jax.ShapeDtypeStruct((ER2, CHUNK), jnp.int32)],
)


def _init_body(pu, pd, nfq, sd, emb, w0a, w0b, w0c, w0d, w0e, b0, wbig,
               out0, outp):
    nfi = nfq[...]
    oh = (lax.broadcasted_iota(jnp.int32, (NBLK, NF), 1)
          == jnp.clip(nfi, 0, MAX_NF)).astype(_f32)
    nemb = jnp.dot(oh, emb[...], preferred_element_type=_f32)
    x = (jnp.dot(pu[...], w0a[...], preferred_element_type=_f32)
         + jnp.dot(pd[...], w0b[...], preferred_element_type=_f32)
         + jnp.dot(nemb, w0c[...], preferred_element_type=_f32)
         + sd[...] * w0d[...]
         + (nfi.astype(_f32) * (1.0 / MAX_NF)) * w0e[...] + b0[...])
    o = jax.nn.relu(x)
    out0[...] = o
    outp[...] = jnp.dot(o, wbig[...], preferred_element_type=_f32)


def _whole(shape):
    return pl.BlockSpec(shape, lambda i: (0,) * len(shape))


_init_call = pl.pallas_call(
    _init_body,
    grid=(N // NBLK,),
    in_specs=[pl.BlockSpec((NBLK, P), lambda i: (i, 0)),
              pl.BlockSpec((NBLK, P), lambda i: (i, 0)),
              pl.BlockSpec((NBLK, 1), lambda i: (i, 0)),
              pl.BlockSpec((NBLK, 1), lambda i: (i, 0)),
              _whole((NF, D)),
              _whole((P, D)), _whole((P, D)), _whole((D, D)),
              _whole((1, D)), _whole((1, D)), _whole((1, D)),
              _whole((D, NF * D))],
    out_specs=[pl.BlockSpec((NBLK, D), lambda i: (i, 0)),
               pl.BlockSpec((NBLK, NF * D), lambda i: (i, 0))],
    out_shape=[jax.ShapeDtypeStruct((N, D), _f32),
               jax.ShapeDtypeStruct((N, NF * D), _f32)],
)


NBUF = 4


def _mp_body(outp_hbm, eidx_hbm, dst_hbm, aggs_hbm,
             zbuf, eidx_v, dst_v, rows_v, agg_sh, isem, gsem, ssem):
    cid = lax.axis_index("c")
    sid = lax.axis_index("s")

    base = cid * (EROWS // NC) + sid * RPW
    idx_cp = pltpu.async_copy(eidx_hbm.at[pl.ds(base, RPW)], eidx_v, isem)
    dst_cp = pltpu.async_copy(dst_hbm.at[pl.ds(base, RPW)], dst_v, isem)

    def zr(i, c):
        zbuf[i, pl.ds(0, 16)] = jnp.zeros((16,), _f32)
        zbuf[i, pl.ds(16, 16)] = jnp.zeros((16,), _f32)
        return c

    lax.fori_loop(0, RPT, zr, 0)
    idx_cp.wait()
    dst_cp.wait()
    for k in range(NBUF):
        pltpu.async_copy(outp_hbm.at[eidx_v.at[k]], rows_v.at[k], gsem.at[k])
    pltpu.sync_copy(zbuf, agg_sh.at[pl.ds(sid * RPT, RPT)])
    plsc.subcore_barrier()

    def round_(r, c):
        j0 = r * NBUF
        for k in range(NBUF):
            j = j0 + k
            pltpu.make_async_copy(
                outp_hbm.at[eidx_v.at[j]], rows_v.at[k], gsem.at[k]).wait()
            pltpu.async_copy(rows_v.at[k], agg_sh.at[dst_v.at[j]], ssem.at[k],
                             add=True)
        for k in range(NBUF):
            j = j0 + k

            @pl.when(j + NBUF < RPW)
            def _():
                pltpu.make_async_copy(
                    rows_v.at[k], agg_sh.at[dst_v.at[j]], ssem.at[k]).wait()
                pltpu.async_copy(outp_hbm.at[eidx_v.at[j + NBUF]],
                                 rows_v.at[k], gsem.at[k])
        return c

    lax.fori_loop(0, RPW // NBUF, round_, 0)
    for k in range(NBUF):
        j = RPW - NBUF + k
        pltpu.make_async_copy(
            rows_v.at[k], agg_sh.at[dst_v.at[j]], ssem.at[k]).wait()
    plsc.subcore_barrier()
    pltpu.sync_copy(agg_sh.at[pl.ds(sid * RPT, RPT)], zbuf)
    pltpu.sync_copy(zbuf, aggs_hbm.at[pl.ds(cid * NROWS + sid * RPT, RPT)])


_mp_call = pl.kernel(
    _mp_body,
    out_type=jax.ShapeDtypeStruct((NC * NROWS, D), _f32),
    mesh=plsc.VectorSubcoreMesh(core_axis_name="c", subcore_axis_name="s",
                                num_cores=NC, num_subcores=NS),
    scratch_types=[pltpu.VMEM((RPT, D), _f32),
                   pltpu.VMEM((RPW, CHUNK), jnp.int32),
                   pltpu.VMEM((RPW, CHUNK), jnp.int32),
                   pltpu.VMEM((NBUF, CHUNK, D), _f32),
                   pltpu.VMEM_SHARED((NROWS, D), _f32),
                   pltpu.SemaphoreType.DMA,
                   pltpu.SemaphoreType.DMA((NBUF,)),
                   pltpu.SemaphoreType.DMA((NBUF,))],
    compiler_params=pltpu.CompilerParams(use_tc_tiling_on_sc=False),
)


def _gru_body(a0, a1, h, cb, wir, wiz, win, bir, biz, bin_, whr, whz, whn,
              bhr, bhz, bhn, wbig, hout, outp):
    hv = h[...]
    m = jax.nn.relu(a0[...] + a1[...] + cb[...])
    r = jax.nn.sigmoid(jnp.dot(m, wir[...], preferred_element_type=_f32) + bir[...]
                       + jnp.dot(hv, whr[...], preferred_element_type=_f32) + bhr[...])
    z = jax.nn.sigmoid(jnp.dot(m, wiz[...], preferred_element_type=_f32) + biz[...]
                       + jnp.dot(hv, whz[...], preferred_element_type=_f32) + bhz[...])
    hn = jnp.dot(hv, whn[...], preferred_element_type=_f32) + bhn[...]
    n_ = jnp.tanh(jnp.dot(m, win[...], preferred_element_type=_f32) + bin_[...] + r * hn)
    hnew = (1.0 - z) * n_ + z * hv
    hout[...] = hnew
    outp[...] = jnp.dot(hnew, wbig[...], preferred_element_type=_f32)


_gru_call = pl.pallas_call(
    _gru_body,
    grid=(N // NBLK,),
    in_specs=[pl.BlockSpec((NBLK, D), lambda i: (i, 0)),
              pl.BlockSpec((NBLK, D), lambda i: (i, 0)),
              pl.BlockSpec((NBLK, D), lambda i: (i, 0)),
              _whole((1, D)),
              _whole((D, D)), _whole((D, D)), _whole((D, D)),
              _whole((1, D)), _whole((1, D)), _whole((1, D)),
              _whole((D, D)), _whole((D, D)), _whole((D, D)),
              _whole((1, D)), _whole((1, D)), _whole((1, D)),
              _whole((D, NF * D))],
    out_specs=[pl.BlockSpec((NBLK, D), lambda i: (i, 0)),
               pl.BlockSpec((NBLK, NF * D), lambda i: (i, 0))],
    out_shape=[jax.ShapeDtypeStruct((N, D), _f32),
               jax.ShapeDtypeStruct((N, NF * D), _f32)],
)


def _s2s_body(h_ref, w0q, w0r, u0, b0, w1g, u1, b1, w2g, u2, b2,
              l1q, l1r, l1b, l2w, l2b, out_ref):
    x = h_ref[...]
    w0qv, w0rv, u0v, b0v = w0q[...], w0r[...], u0[...], b0[...]
    w1v, u1v, b1v = w1g[...], u1[...], b1[...]
    w2v, u2v, b2v = w2g[...], u2[...], b2[...]
    zero = jnp.zeros((1, D), _f32)

    def gates(xs, ws, u, b, hp):
        out = []
        for g in range(4):
            acc = b[g] + jnp.dot(hp, u[g], preferred_element_type=_f32)
            for xx, w in zip(xs, ws):
                acc = acc + jnp.dot(xx, w[g], preferred_element_type=_f32)
            out.append(acc)
        return out

    def it(t, carry):
        q, r, h0, c0, h1, c1, h2, c2 = carry
        gi, gf, gg, go = gates([q, r], [w0qv, w0rv], u0v, b0v, h0)
        c0 = jax.nn.sigmoid(gf) * c0 + jax.nn.sigmoid(gi) * jnp.tanh(gg)
        h0 = jax.nn.sigmoid(go) * jnp.tanh(c0)
        gi, gf, gg, go = gates([h0], [w1v], u1v, b1v, h1)
        c1 = jax.nn.sigmoid(gf) * c1 + jax.nn.sigmoid(gi) * jnp.tanh(gg)
        h1 = jax.nn.sigmoid(go) * jnp.tanh(c1)
        gi, gf, gg, go = gates([h1], [w2v], u2v, b2v, h2)
        c2 = jax.nn.sigmoid(gf) * c2 + jax.nn.sigmoid(gi) * jnp.tanh(gg)
        h2 = jax.nn.sigmoid(go) * jnp.tanh(c2)
        q = h2
        e = jnp.sum(x * q, axis=1, keepdims=True)
        a = jnp.exp(e - jnp.max(e))
        r = jnp.sum(a * x, axis=0, keepdims=True) / jnp.sum(a)
        return (q, r, h0, c0, h1, c1, h2, c2)

    q, r = lax.fori_loop(0, T_S2S, it, (zero,) * 8)[:2]
    y = jax.nn.relu(jnp.dot(q, l1q[...], preferred_element_type=_f32)
                    + jnp.dot(r, l1r[...], preferred_element_type=_f32) + l1b[...])
    y = jnp.dot(y, l2w[...], preferred_element_type=_f32) + l2b[...]
    out_ref[...] = jnp.broadcast_to(y, (8, D))


_s2s_call = pl.pallas_call(
    _s2s_body,
    out_shape=jax.ShapeDtypeStruct((8, D), _f32),
)


def kernel(pos_undirected, pos_directed, params, nfreq, seed, efreq, edge_index):
    p = params
    src = edge_index[0].astype(jnp.int32)
    dst = edge_index[1].astype(jnp.int32)
    efq = efreq.astype(jnp.int32)

    ef_feat = jnp.concatenate(
        [p['edge_freq_emb'], (jnp.arange(NF, dtype=_f32) / MAX_EF)[:, None]], axis=1)
    ewtab, eidx2d = _const_call(
        ef_feat, p['edge_W1'], p['edge_b1'][None], p['edge_W2'], p['edge_b2'][None],
        src.reshape(ER2, CHUNK), efq.reshape(ER2, CHUNK))
    wbig = ewtab.reshape(NF, D, D).transpose(1, 0, 2).reshape(D, NF * D)
    eidx = jnp.concatenate(
        [eidx2d.reshape(-1), jnp.zeros((EPAD - E,), jnp.int32)]).reshape(EROWS, CHUNK)
    dstp = jnp.concatenate(
        [dst, jnp.full((EPAD - E,), N, jnp.int32)]).reshape(EROWS, CHUNK)

    w0 = p['lin0_W']
    h, outp = _init_call(
        pos_undirected, pos_directed, nfreq.astype(jnp.int32)[:, None],
        seed.astype(_f32)[:, None], p['node_freq_emb'],
        w0[0:P], w0[P:2 * P], w0[2 * P:2 * P + D],
        w0[2 * P + D:2 * P + D + 1], w0[2 * P + D + 1:2 * P + D + 2],
        p['lin0_b'][None], wbig)

    wih, bih, whh, bhh = p['gru_Wih'], p['gru_bih'], p['gru_Whh'], p['gru_bhh']
    for _ in range(T_MP):
        aggs = _mp_call(outp.reshape(N * NF, D), eidx, dstp)
        h, outp = _gru_call(
            aggs[0:N], aggs[NROWS:NROWS + N], h, p['conv_bias'][None],
            wih[:, 0:D], wih[:, D:2 * D], wih[:, 2 * D:3 * D],
            bih[None, 0:D], bih[None, D:2 * D], bih[None, 2 * D:3 * D],
            whh[:, 0:D], whh[:, D:2 * D], whh[:, 2 * D:3 * D],
            bhh[None, 0:D], bhh[None, D:2 * D], bhh[None, 2 * D:3 * D],
            wbig)

    def stkw(wm, r0, r1):
        return jnp.stack([wm[r0:r1, g * D:(g + 1) * D] for g in range(4)])

    def stkb(l):
        b = p['lstm%d_bih' % l] + p['lstm%d_bhh' % l]
        return jnp.stack([b[None, g * D:(g + 1) * D] for g in range(4)])

    y8 = _s2s_call(
        h,
        stkw(p['lstm0_Wih'], 0, D), stkw(p['lstm0_Wih'], D, 2 * D),
        stkw(p['lstm0_Whh'], 0, D), stkb(0),
        stkw(p['lstm1_Wih'], 0, D), stkw(p['lstm1_Whh'], 0, D), stkb(1),
        stkw(p['lstm2_Wih'], 0, D), stkw(p['lstm2_Whh'], 0, D), stkb(2),
        p['lin1_W'][0:D], p['lin1_W'][D:2 * D], p['lin1_b'][None],
        p['lin2_W'], p['lin2_b'][None])
    return y8[0:1]


# speedup vs baseline: 15.8568x; 1.5277x over previous
import functools

import jax
import jax.numpy as jnp
from jax import lax
from jax.experimental import pallas as pl
from jax.experimental.pallas import tpu as pltpu
from jax.experimental.pallas import tpu_sc as plsc

N = 10000
E = 160000
P = 16
D = 32
MAX_NF = 8
MAX_EF = 8
NF = MAX_EF + 1
T_MP = 3
T_S2S = 6

NC, NS = 2, 16
CHUNK = 125
EROWS = E // CHUNK
RPW = EROWS // (NC * NS)
NROWS = N
RPT = NROWS // NS
NBLK = 2000

_f32 = jnp.float32


def _const_body(ef, w1, b1, w2, b2, src, efq, ewtab, eidx):
    v = jax.nn.relu(jnp.dot(ef[...], w1[...], preferred_element_type=_f32) + b1[...])
    ewtab[...] = jnp.dot(v, w2[...], preferred_element_type=_f32) + b2[...]
    eidx[...] = src[...] * NF + jnp.clip(efq[...], 0, MAX_EF)


_const_call = pl.pallas_call(
    _const_body,
    out_shape=[jax.ShapeDtypeStruct((NF, D * D), _f32),
               jax.ShapeDtypeStruct((EROWS, CHUNK), jnp.int32)],
)


def _init_body(pu, pd, nfq, sd, emb, w0a, w0b, w0c, w0d, w0e, b0, wbig,
               out0, outp):
    nfi = nfq[...]
    oh = (lax.broadcasted_iota(jnp.int32, (NBLK, NF), 1)
          == jnp.clip(nfi, 0, MAX_NF)).astype(_f32)
    nemb = jnp.dot(oh, emb[...], preferred_element_type=_f32)
    x = (jnp.dot(pu[...], w0a[...], preferred_element_type=_f32)
         + jnp.dot(pd[...], w0b[...], preferred_element_type=_f32)
         + jnp.dot(nemb, w0c[...], preferred_element_type=_f32)
         + sd[...] * w0d[...]
         + (nfi.astype(_f32) * (1.0 / MAX_NF)) * w0e[...] + b0[...])
    o = jax.nn.relu(x)
    out0[...] = o
    outp[...] = jnp.dot(o, wbig[...], preferred_element_type=_f32)


def _whole(shape):
    return pl.BlockSpec(shape, lambda i: (0,) * len(shape))


_init_call = pl.pallas_call(
    _init_body,
    grid=(N // NBLK,),
    in_specs=[pl.BlockSpec((NBLK, P), lambda i: (i, 0)),
              pl.BlockSpec((NBLK, P), lambda i: (i, 0)),
              pl.BlockSpec((NBLK, 1), lambda i: (i, 0)),
              pl.BlockSpec((NBLK, 1), lambda i: (i, 0)),
              _whole((NF, D)),
              _whole((P, D)), _whole((P, D)), _whole((D, D)),
              _whole((1, D)), _whole((1, D)), _whole((1, D)),
              _whole((D, NF * D))],
    out_specs=[pl.BlockSpec((NBLK, D), lambda i: (i, 0)),
               pl.BlockSpec((NBLK, NF * D), lambda i: (i, 0))],
    out_shape=[jax.ShapeDtypeStruct((N, D), _f32),
               jax.ShapeDtypeStruct((N, NF * D), _f32)],
)


NBUF = 4


def _mp_body(outp_hbm, eidx_hbm, dst_hbm, aggs_hbm,
             zbuf, eidx_v, dst_v, rows_v, agg_sh, isem, gsem, ssem):
    cid = lax.axis_index("c")
    sid = lax.axis_index("s")

    base = cid * (EROWS // NC) + sid * RPW
    idx_cp = pltpu.async_copy(eidx_hbm.at[pl.ds(base, RPW)], eidx_v, isem)
    dst_cp = pltpu.async_copy(dst_hbm.at[pl.ds(base, RPW)], dst_v, isem)

    def zr(i, c):
        zbuf[i, pl.ds(0, 16)] = jnp.zeros((16,), _f32)
        zbuf[i, pl.ds(16, 16)] = jnp.zeros((16,), _f32)
        return c

    lax.fori_loop(0, RPT, zr, 0)
    idx_cp.wait()
    dst_cp.wait()
    for k in range(NBUF):
        pltpu.async_copy(outp_hbm.at[eidx_v.at[k]], rows_v.at[k], gsem.at[k])
    pltpu.sync_copy(zbuf, agg_sh.at[pl.ds(sid * RPT, RPT)])
    plsc.subcore_barrier()

    def round_(r, c):
        j0 = r * NBUF
        for k in range(NBUF):
            j = j0 + k
            pltpu.make_async_copy(
                outp_hbm.at[eidx_v.at[j]], rows_v.at[k], gsem.at[k]).wait()
            pltpu.async_copy(rows_v.at[k], agg_sh.at[dst_v.at[j]], ssem.at[k],
                             add=True)
        for k in range(NBUF):
            j = j0 + k

            @pl.when(j + NBUF < RPW)
            def _():
                pltpu.make_async_copy(
                    rows_v.at[k], agg_sh.at[dst_v.at[j]], ssem.at[k]).wait()
                pltpu.async_copy(outp_hbm.at[eidx_v.at[j + NBUF]],
                                 rows_v.at[k], gsem.at[k])
        return c

    lax.fori_loop(0, RPW // NBUF, round_, 0)
    for k in range(NBUF):
        j = RPW - NBUF + k
        pltpu.make_async_copy(
            rows_v.at[k], agg_sh.at[dst_v.at[j]], ssem.at[k]).wait()
    plsc.subcore_barrier()
    pltpu.sync_copy(agg_sh.at[pl.ds(sid * RPT, RPT)], zbuf)
    pltpu.sync_copy(zbuf, aggs_hbm.at[pl.ds(cid * NROWS + sid * RPT, RPT)])


_mp_call = pl.kernel(
    _mp_body,
    out_type=jax.ShapeDtypeStruct((NC * NROWS, D), _f32),
    mesh=plsc.VectorSubcoreMesh(core_axis_name="c", subcore_axis_name="s",
                                num_cores=NC, num_subcores=NS),
    scratch_types=[pltpu.VMEM((RPT, D), _f32),
                   pltpu.VMEM((RPW, CHUNK), jnp.int32),
                   pltpu.VMEM((RPW, CHUNK), jnp.int32),
                   pltpu.VMEM((NBUF, CHUNK, D), _f32),
                   pltpu.VMEM_SHARED((NROWS, D), _f32),
                   pltpu.SemaphoreType.DMA,
                   pltpu.SemaphoreType.DMA((NBUF,)),
                   pltpu.SemaphoreType.DMA((NBUF,))],
    compiler_params=pltpu.CompilerParams(use_tc_tiling_on_sc=False),
)


def _gru_body(a0, a1, h, cb, wir, wiz, win, bir, biz, bin_, whr, whz, whn,
              bhr, bhz, bhn, wbig, hout, outp):
    hv = h[...]
    m = jax.nn.relu(a0[...] + a1[...] + cb[...])
    r = jax.nn.sigmoid(jnp.dot(m, wir[...], preferred_element_type=_f32) + bir[...]
                       + jnp.dot(hv, whr[...], preferred_element_type=_f32) + bhr[...])
    z = jax.nn.sigmoid(jnp.dot(m, wiz[...], preferred_element_type=_f32) + biz[...]
                       + jnp.dot(hv, whz[...], preferred_element_type=_f32) + bhz[...])
    hn = jnp.dot(hv, whn[...], preferred_element_type=_f32) + bhn[...]
    n_ = jnp.tanh(jnp.dot(m, win[...], preferred_element_type=_f32) + bin_[...] + r * hn)
    hnew = (1.0 - z) * n_ + z * hv
    hout[...] = hnew
    outp[...] = jnp.dot(hnew, wbig[...], preferred_element_type=_f32)


_gru_call = pl.pallas_call(
    _gru_body,
    grid=(N // NBLK,),
    in_specs=[pl.BlockSpec((NBLK, D), lambda i: (i, 0)),
              pl.BlockSpec((NBLK, D), lambda i: (i + NROWS // NBLK, 0)),
              pl.BlockSpec((NBLK, D), lambda i: (i, 0)),
              _whole((1, D)),
              _whole((D, D)), _whole((D, D)), _whole((D, D)),
              _whole((1, D)), _whole((1, D)), _whole((1, D)),
              _whole((D, D)), _whole((D, D)), _whole((D, D)),
              _whole((1, D)), _whole((1, D)), _whole((1, D)),
              _whole((D, NF * D))],
    out_specs=[pl.BlockSpec((NBLK, D), lambda i: (i, 0)),
               pl.BlockSpec((NBLK, NF * D), lambda i: (i, 0))],
    out_shape=[jax.ShapeDtypeStruct((N, D), _f32),
               jax.ShapeDtypeStruct((N, NF * D), _f32)],
)


def _s2s_body(h_ref, w0q, w0r, u0, b0, w1g, u1, b1, w2g, u2, b2,
              l1q, l1r, l1b, l2w, l2b, out_ref):
    x = h_ref[...]
    w0qv, w0rv, u0v, b0v = w0q[...], w0r[...], u0[...], b0[...]
    w1v, u1v, b1v = w1g[...], u1[...], b1[...]
    w2v, u2v, b2v = w2g[...], u2[...], b2[...]
    zero = jnp.zeros((1, D), _f32)

    def gates(xs, ws, u, b, hp):
        out = []
        for g in range(4):
            acc = b[g] + jnp.dot(hp, u[g], preferred_element_type=_f32)
            for xx, w in zip(xs, ws):
                acc = acc + jnp.dot(xx, w[g], preferred_element_type=_f32)
            out.append(acc)
        return out

    def it(t, carry):
        q, r, h0, c0, h1, c1, h2, c2 = carry
        gi, gf, gg, go = gates([q, r], [w0qv, w0rv], u0v, b0v, h0)
        c0 = jax.nn.sigmoid(gf) * c0 + jax.nn.sigmoid(gi) * jnp.tanh(gg)
        h0 = jax.nn.sigmoid(go) * jnp.tanh(c0)
        gi, gf, gg, go = gates([h0], [w1v], u1v, b1v, h1)
        c1 = jax.nn.sigmoid(gf) * c1 + jax.nn.sigmoid(gi) * jnp.tanh(gg)
        h1 = jax.nn.sigmoid(go) * jnp.tanh(c1)
        gi, gf, gg, go = gates([h1], [w2v], u2v, b2v, h2)
        c2 = jax.nn.sigmoid(gf) * c2 + jax.nn.sigmoid(gi) * jnp.tanh(gg)
        h2 = jax.nn.sigmoid(go) * jnp.tanh(c2)
        q = h2
        e = jnp.sum(x * q, axis=1, keepdims=True)
        a = jnp.exp(e - jnp.max(e))
        r = jnp.sum(a * x, axis=0, keepdims=True) / jnp.sum(a)
        return (q, r, h0, c0, h1, c1, h2, c2)

    q, r = lax.fori_loop(0, T_S2S, it, (zero,) * 8)[:2]
    y = jax.nn.relu(jnp.dot(q, l1q[...], preferred_element_type=_f32)
                    + jnp.dot(r, l1r[...], preferred_element_type=_f32) + l1b[...])
    y = jnp.dot(y, l2w[...], preferred_element_type=_f32) + l2b[...]
    out_ref[...] = jnp.broadcast_to(y, (8, D))


_s2s_call = pl.pallas_call(
    _s2s_body,
    out_shape=jax.ShapeDtypeStruct((8, D), _f32),
)


def kernel(pos_undirected, pos_directed, params, nfreq, seed, efreq, edge_index):
    p = params
    src = edge_index[0].astype(jnp.int32)
    dst = edge_index[1].astype(jnp.int32)
    efq = efreq.astype(jnp.int32)

    ef_feat = jnp.concatenate(
        [p['edge_freq_emb'], (jnp.arange(NF, dtype=_f32) / MAX_EF)[:, None]], axis=1)
    ewtab, eidx = _const_call(
        ef_feat, p['edge_W1'], p['edge_b1'][None], p['edge_W2'], p['edge_b2'][None],
        src.reshape(EROWS, CHUNK), efq.reshape(EROWS, CHUNK))
    wbig = ewtab.reshape(NF, D, D).transpose(1, 0, 2).reshape(D, NF * D)
    dstp = dst.reshape(EROWS, CHUNK)

    w0 = p['lin0_W']
    h, outp = _init_call(
        pos_undirected, pos_directed, nfreq.astype(jnp.int32)[:, None],
        seed.astype(_f32)[:, None], p['node_freq_emb'],
        w0[0:P], w0[P:2 * P], w0[2 * P:2 * P + D],
        w0[2 * P + D:2 * P + D + 1], w0[2 * P + D + 1:2 * P + D + 2],
        p['lin0_b'][None], wbig)

    wih, bih, whh, bhh = p['gru_Wih'], p['gru_bih'], p['gru_Whh'], p['gru_bhh']
    for _ in range(T_MP):
        aggs = _mp_call(outp.reshape(N * NF, D), eidx, dstp)
        h, outp = _gru_call(
            aggs, aggs, h, p['conv_bias'][None],
            wih[:, 0:D], wih[:, D:2 * D], wih[:, 2 * D:3 * D],
            bih[None, 0:D], bih[None, D:2 * D], bih[None, 2 * D:3 * D],
            whh[:, 0:D], whh[:, D:2 * D], whh[:, 2 * D:3 * D],
            bhh[None, 0:D], bhh[None, D:2 * D], bhh[None, 2 * D:3 * D],
            wbig)

    def stkw(wm, r0, r1):
        return jnp.stack([wm[r0:r1, g * D:(g + 1) * D] for g in range(4)])

    def stkb(l):
        b = p['lstm%d_bih' % l] + p['lstm%d_bhh' % l]
        return jnp.stack([b[None, g * D:(g + 1) * D] for g in range(4)])

    y8 = _s2s_call(
        h,
        stkw(p['lstm0_Wih'], 0, D), stkw(p['lstm0_Wih'], D, 2 * D),
        stkw(p['lstm0_Whh'], 0, D), stkb(0),
        stkw(p['lstm1_Wih'], 0, D), stkw(p['lstm1_Whh'], 0, D), stkb(1),
        stkw(p['lstm2_Wih'], 0, D), stkw(p['lstm2_Whh'], 0, D), stkb(2),
        p['lin1_W'][0:D], p['lin1_W'][D:2 * D], p['lin1_b'][None],
        p['lin2_W'], p['lin2_b'][None])
    return y8[0:1]
